# baseline (speedup 1.0000x reference)
import functools

import jax
import jax.numpy as jnp
from jax import lax
from jax.experimental import pallas as pl
from jax.experimental.pallas import tpu as pltpu
from jax.experimental.pallas import tpu_sc as plsc

B = 4096
F = 32
W = 20
NLISTS = 12
K_LIST = NLISTS * W * F
NUM_DENSE_ = 2 * F + 12 * F * W + 2 + 26
H_REAL = NUM_DENSE_ // 2
HEAD_USED = 2 * F + 2 + 26
HEAD_PAD = 128
HBLK = 512
BBLK = 1024
NH = -(-H_REAL // HBLK)
NBB = B // BBLK

NC = 2
NS = 16
NW = NC * NS
NROWS = B * NLISTS * W
ROWS_PER_TILE = NROWS // NW
CHUNK = 1024
SUB = CHUNK // 128
NCH = ROWS_PER_TILE // CHUNK
UCHUNK = B // NW

@functools.cache
def _sc_gather_call():
    mesh = plsc.VectorSubcoreMesh(core_axis_name="c", subcore_axis_name="s")

    @functools.partial(
        pl.kernel,
        mesh=mesh,
        compiler_params=pltpu.CompilerParams(use_tc_tiling_on_sc=False),
        out_type=[
            jax.ShapeDtypeStruct((NROWS, F), jnp.float32),
            jax.ShapeDtypeStruct((B, F), jnp.float32),
            jax.ShapeDtypeStruct((B, F), jnp.float32),
        ],
        scratch_types=[
            pltpu.VMEM((SUB, 128), jnp.int32),
            pltpu.VMEM((CHUNK, F), jnp.float32),
            pltpu.VMEM((UCHUNK,), jnp.int32),
            pltpu.VMEM((UCHUNK, F), jnp.float32),
            pltpu.SemaphoreType.DMA,
        ],
    )
    def _sc_gather(tab_hbm, idx_hbm, uid_hbm, iid_hbm, utab_hbm, itab_hbm,
                   out_hbm, ue_hbm, ie_hbm,
                   idx_v, rows_v, sid_v, srow_v, sem):
        wid = lax.axis_index("s") * NC + lax.axis_index("c")
        base = wid * ROWS_PER_TILE

        def chunk_body(g, carry):
            off = base + g * CHUNK
            row0 = wid * (ROWS_PER_TILE // 128) + g * SUB
            pltpu.sync_copy(idx_hbm.at[pl.ds(row0, SUB)], idx_v)
            cps = [
                pltpu.async_copy(tab_hbm.at[idx_v.at[k]],
                                 rows_v.at[pl.ds(k * 128, 128)], sem)
                for k in range(SUB)
            ]
            for cp in cps:
                cp.wait()
            pltpu.sync_copy(rows_v, out_hbm.at[pl.ds(off, CHUNK)])
            return carry

        lax.fori_loop(0, NCH, chunk_body, 0)

        ub = wid * UCHUNK
        pltpu.sync_copy(uid_hbm.at[pl.ds(ub, UCHUNK)], sid_v)
        pltpu.async_copy(utab_hbm.at[sid_v], srow_v, sem).wait()
        pltpu.sync_copy(srow_v, ue_hbm.at[pl.ds(ub, UCHUNK)])
        pltpu.sync_copy(iid_hbm.at[pl.ds(ub, UCHUNK)], sid_v)
        pltpu.async_copy(itab_hbm.at[sid_v], srow_v, sem).wait()
        pltpu.sync_copy(srow_v, ie_hbm.at[pl.ds(ub, UCHUNK)])

    return _sc_gather


def _cast_body(x_ref, o_ref):
    o_ref[...] = x_ref[...].astype(jnp.bfloat16)


_cast_call = pl.pallas_call(
    _cast_body,
    grid=(B // 512,),
    in_specs=[pl.BlockSpec((512, K_LIST), lambda i: (i, 0))],
    out_specs=pl.BlockSpec((512, K_LIST), lambda i: (i, 0)),
    out_shape=jax.ShapeDtypeStruct((B, K_LIST), jnp.bfloat16),
    compiler_params=pltpu.CompilerParams(
        dimension_semantics=("arbitrary",)),
)


def _mlp_body(head_ref, x_ref, w1h_ref, w1p_ref, b1_ref, w2_ref, b2_ref,
              wo_ref, bo_ref, out_ref, acc_ref):
    j = pl.program_id(1)

    @pl.when(j == 0)
    def _init():
        acc_ref[...] = jnp.zeros_like(acc_ref)

    h = jnp.dot(x_ref[...], w1p_ref[...],
                preferred_element_type=jnp.float32)
    h += jnp.dot(head_ref[...], w1h_ref[...],
                 preferred_element_type=jnp.float32)
    h = jnp.maximum(h + b1_ref[...], 0.0)
    cmask = (j * HBLK + lax.broadcasted_iota(jnp.int32, (1, HBLK), 1)
             ) < H_REAL
    h = jnp.where(cmask, h, 0.0)
    rmask = (j * HBLK + lax.broadcasted_iota(jnp.int32, (HBLK, 1), 0)
             ) < H_REAL
    w2m = jnp.where(rmask, w2_ref[...], 0.0)
    acc_ref[...] += jnp.dot(h, w2m, preferred_element_type=jnp.float32)

    @pl.when(j == NH - 1)
    def _fin():
        z = acc_ref[...] + b2_ref[...]
        logit = jnp.dot(z, wo_ref[...],
                        preferred_element_type=jnp.float32) + bo_ref[...]
        out_ref[...] = jax.nn.sigmoid(logit)


_mlp_call = pl.pallas_call(
    _mlp_body,
    grid=(NBB, NH),
    in_specs=[
        pl.BlockSpec((BBLK, HEAD_PAD), lambda i, j: (i, 0)),
        pl.BlockSpec((BBLK, K_LIST), lambda i, j: (i, 0)),
        pl.BlockSpec((HEAD_PAD, HBLK), lambda i, j: (0, j)),
        pl.BlockSpec((K_LIST, HBLK), lambda i, j: (0, j)),
        pl.BlockSpec((1, HBLK), lambda i, j: (0, j)),
        pl.BlockSpec((HBLK, F), lambda i, j: (j, 0)),
        pl.BlockSpec((1, F), lambda i, j: (0, 0)),
        pl.BlockSpec((F, 1), lambda i, j: (0, 0)),
        pl.BlockSpec((1, 1), lambda i, j: (0, 0)),
    ],
    out_specs=pl.BlockSpec((BBLK, 1), lambda i, j: (i, 0)),
    out_shape=jax.ShapeDtypeStruct((B, 1), jnp.float32),
    scratch_shapes=[pltpu.VMEM((BBLK, F), jnp.float32)],
    compiler_params=pltpu.CompilerParams(
        dimension_semantics=("arbitrary", "arbitrary")),
)

_LIST_OFFS = (0, 11, 11, 11, 11, 11, 11,
              100012, 100012, 100012, 100012, 100012)


def kernel(user_ids, item_ids, price, platform_idx, device_idx, pos_item_idx,
           list_action_type_idx, list_clickout_item_idx,
           list_interaction_item_image_idx, list_interaction_item_info_idx,
           list_interaction_item_rating_idx, list_interaction_item_deals_idx,
           list_search_for_item_idx, list_search_for_poi,
           list_change_of_sort_order, list_search_for_destination,
           list_filter_selection, list_current_filters, list_metadata,
           user_table, item_table, action_table, word_table,
           W1, b1, W2, b2, Wo, bo):
    lists = [list_action_type_idx, list_clickout_item_idx,
             list_interaction_item_image_idx, list_interaction_item_info_idx,
             list_interaction_item_rating_idx, list_interaction_item_deals_idx,
             list_search_for_item_idx, list_search_for_poi,
             list_change_of_sort_order, list_search_for_destination,
             list_filter_selection, list_current_filters]
    offs = jnp.asarray(_LIST_OFFS, dtype=jnp.int32)
    idx_all = (jnp.stack(lists, axis=2) + offs[None, None, :]).reshape(-1, 128)
    tab = jnp.concatenate([action_table, item_table, word_table], axis=0)

    x3, ue, ie = _sc_gather_call()(tab, idx_all, user_ids, item_ids,
                                   user_table, item_table)
    xl = _cast_call(x3.reshape(B, K_LIST))

    head = jnp.concatenate(
        [ue, ie, pos_item_idx.astype(jnp.float32)[:, None], price[:, None],
         list_metadata,
         jnp.zeros((B, HEAD_PAD - HEAD_USED), jnp.float32)],
        axis=1).astype(jnp.bfloat16)

    w1h = jnp.pad(W1[:HEAD_USED],
                  ((0, HEAD_PAD - HEAD_USED), (0, 0))).astype(jnp.bfloat16)
    w1p = W1[HEAD_USED:].astype(jnp.bfloat16)

    return _mlp_call(head, xl, w1h, w1p, b1.reshape(1, H_REAL), W2,
                     b2.reshape(1, F), Wo, bo.reshape(1, 1))


# speedup vs baseline: 1.0415x; 1.0415x over previous
import functools

import jax
import jax.numpy as jnp
from jax import lax
from jax.experimental import pallas as pl
from jax.experimental.pallas import tpu as pltpu
from jax.experimental.pallas import tpu_sc as plsc

B = 4096
F = 32
W = 20
NLISTS = 12
K_LIST = NLISTS * W * F
NUM_DENSE_ = 2 * F + 12 * F * W + 2 + 26
H_REAL = NUM_DENSE_ // 2
HEAD_USED = 2 * F + 2 + 26
N_ACT_ = 11
HEAD_PAD = 128
HBLK = 512
BBLK = 1024
NH = -(-H_REAL // HBLK)
NBB = B // BBLK

NC = 2
NS = 16
NW = NC * NS
NROWS = B * NLISTS * W
ROWS_PER_TILE = NROWS // NW
CHUNK = 1024
SUB = CHUNK // 128
NCH = ROWS_PER_TILE // CHUNK
UCHUNK = B // NW

@functools.cache
def _sc_gather_call():
    mesh = plsc.VectorSubcoreMesh(core_axis_name="c", subcore_axis_name="s")

    @functools.partial(
        pl.kernel,
        mesh=mesh,
        compiler_params=pltpu.CompilerParams(use_tc_tiling_on_sc=False),
        out_type=[
            jax.ShapeDtypeStruct((NROWS, F), jnp.float32),
            jax.ShapeDtypeStruct((B, F), jnp.float32),
            jax.ShapeDtypeStruct((B, F), jnp.float32),
        ],
        scratch_types=[
            pltpu.VMEM((SUB, 128), jnp.int32),
            pltpu.VMEM((SUB, 128), jnp.int32),
            pltpu.VMEM((CHUNK, F), jnp.float32),
            pltpu.VMEM((CHUNK, F), jnp.float32),
            pltpu.VMEM((UCHUNK,), jnp.int32),
            pltpu.VMEM((UCHUNK, F), jnp.float32),
            pltpu.SemaphoreType.DMA,
            pltpu.SemaphoreType.DMA,
            pltpu.SemaphoreType.DMA,
        ],
    )
    def _sc_gather(tab_hbm, idx_hbm, uid_hbm, iid_hbm, utab_hbm, itab_hbm,
                   out_hbm, ue_hbm, ie_hbm,
                   idx0, idx1, buf0, buf1, sid_v, srow_v, sem0, sem1, sem):
        wid = lax.axis_index("s") * NC + lax.axis_index("c")
        base = wid * ROWS_PER_TILE
        rowbase = wid * (ROWS_PER_TILE // 128)

        def load_idx(c, idxv):
            pltpu.sync_copy(idx_hbm.at[pl.ds(rowbase + c * SUB, SUB)], idxv)

        def fire(idxv, buf, sm):
            for k in range(SUB):
                pltpu.async_copy(tab_hbm.at[idxv.at[k]],
                                 buf.at[pl.ds(k * 128, 128)], sm)

        def drain(idxv, buf, sm):
            for k in range(SUB):
                pltpu.make_async_copy(tab_hbm.at[idxv.at[k]],
                                      buf.at[pl.ds(k * 128, 128)], sm).wait()

        def write(c, buf):
            pltpu.sync_copy(buf, out_hbm.at[pl.ds(base + c * CHUNK, CHUNK)])

        def body(g, carry):
            load_idx(g, idx0)
            fire(idx0, buf0, sem0)
            drain(idx0, buf0, sem0)
            write(g, buf0)
            return carry

        lax.fori_loop(0, NCH, body, 0)

        ub = wid * UCHUNK
        pltpu.sync_copy(uid_hbm.at[pl.ds(ub, UCHUNK)], sid_v)
        pltpu.async_copy(utab_hbm.at[sid_v], srow_v, sem).wait()
        pltpu.sync_copy(srow_v, ue_hbm.at[pl.ds(ub, UCHUNK)])
        pltpu.sync_copy(iid_hbm.at[pl.ds(ub, UCHUNK)], sid_v)
        pltpu.async_copy(itab_hbm.at[sid_v], srow_v, sem).wait()
        pltpu.sync_copy(srow_v, ie_hbm.at[pl.ds(ub, UCHUNK)])

    return _sc_gather


def _cast_body(x_ref, o_ref):
    o_ref[...] = x_ref[...].astype(jnp.bfloat16)


_cast_call = pl.pallas_call(
    _cast_body,
    grid=(B // 512,),
    in_specs=[pl.BlockSpec((512, K_LIST), lambda i: (i, 0))],
    out_specs=pl.BlockSpec((512, K_LIST), lambda i: (i, 0)),
    out_shape=jax.ShapeDtypeStruct((B, K_LIST), jnp.bfloat16),
    compiler_params=pltpu.CompilerParams(
        dimension_semantics=("arbitrary",)),
)


def _mlp_body(head_ref, x_ref, w1h_ref, w1p_ref, b1_ref, w2_ref, b2_ref,
              wo_ref, bo_ref, out_ref, acc_ref):
    j = pl.program_id(1)

    @pl.when(j == 0)
    def _init():
        acc_ref[...] = jnp.zeros_like(acc_ref)

    h = jnp.dot(x_ref[...], w1p_ref[...],
                preferred_element_type=jnp.float32)
    h += jnp.dot(head_ref[...], w1h_ref[...],
                 preferred_element_type=jnp.float32)
    h = jnp.maximum(h + b1_ref[...], 0.0)
    cmask = (j * HBLK + lax.broadcasted_iota(jnp.int32, (1, HBLK), 1)
             ) < H_REAL
    h = jnp.where(cmask, h, 0.0)
    rmask = (j * HBLK + lax.broadcasted_iota(jnp.int32, (HBLK, 1), 0)
             ) < H_REAL
    w2m = jnp.where(rmask, w2_ref[...], 0.0)
    acc_ref[...] += jnp.dot(h, w2m, preferred_element_type=jnp.float32)

    @pl.when(j == NH - 1)
    def _fin():
        z = acc_ref[...] + b2_ref[...]
        logit = jnp.dot(z, wo_ref[...],
                        preferred_element_type=jnp.float32) + bo_ref[...]
        out_ref[...] = jax.nn.sigmoid(logit)


_mlp_call = pl.pallas_call(
    _mlp_body,
    grid=(NBB, NH),
    in_specs=[
        pl.BlockSpec((BBLK, HEAD_PAD), lambda i, j: (i, 0)),
        pl.BlockSpec((BBLK, K_LIST), lambda i, j: (i, 0)),
        pl.BlockSpec((HEAD_PAD, HBLK), lambda i, j: (0, j)),
        pl.BlockSpec((K_LIST, HBLK), lambda i, j: (0, j)),
        pl.BlockSpec((1, HBLK), lambda i, j: (0, j)),
        pl.BlockSpec((HBLK, F), lambda i, j: (j, 0)),
        pl.BlockSpec((1, F), lambda i, j: (0, 0)),
        pl.BlockSpec((F, 1), lambda i, j: (0, 0)),
        pl.BlockSpec((1, 1), lambda i, j: (0, 0)),
    ],
    out_specs=pl.BlockSpec((BBLK, 1), lambda i, j: (i, 0)),
    out_shape=jax.ShapeDtypeStruct((B, 1), jnp.float32),
    scratch_shapes=[pltpu.VMEM((BBLK, F), jnp.float32)],
    compiler_params=pltpu.CompilerParams(
        dimension_semantics=("arbitrary", "arbitrary")),
)

TRC = 2048
TROWS = 99 * TRC
_LIST_OFFS = (0, TRC, TRC, TRC, TRC, TRC, TRC,
              102400, 102400, 102400, 102400, 102400)


def _tab_body(act_ref, item_ref, word_ref, out_ref):
    i = pl.program_id(0)
    out_ref[...] = jnp.where(
        i < 1, act_ref[...],
        jnp.where(i < 50, item_ref[...], word_ref[...]))


_tab_call = pl.pallas_call(
    _tab_body,
    grid=(99,),
    in_specs=[
        pl.BlockSpec((TRC, F), lambda i: (0, 0)),
        pl.BlockSpec((TRC, F), lambda i: (jnp.clip(i - 1, 0, 48), 0)),
        pl.BlockSpec((TRC, F), lambda i: (jnp.clip(i - 50, 0, 48), 0)),
    ],
    out_specs=pl.BlockSpec((TRC, F), lambda i: (i, 0)),
    out_shape=jax.ShapeDtypeStruct((TROWS, F), jnp.float32),
    compiler_params=pltpu.CompilerParams(
        dimension_semantics=("arbitrary",)),
)


def kernel(user_ids, item_ids, price, platform_idx, device_idx, pos_item_idx,
           list_action_type_idx, list_clickout_item_idx,
           list_interaction_item_image_idx, list_interaction_item_info_idx,
           list_interaction_item_rating_idx, list_interaction_item_deals_idx,
           list_search_for_item_idx, list_search_for_poi,
           list_change_of_sort_order, list_search_for_destination,
           list_filter_selection, list_current_filters, list_metadata,
           user_table, item_table, action_table, word_table,
           W1, b1, W2, b2, Wo, bo):
    lists = [list_action_type_idx, list_clickout_item_idx,
             list_interaction_item_image_idx, list_interaction_item_info_idx,
             list_interaction_item_rating_idx, list_interaction_item_deals_idx,
             list_search_for_item_idx, list_search_for_poi,
             list_change_of_sort_order, list_search_for_destination,
             list_filter_selection, list_current_filters]
    offs = jnp.asarray(_LIST_OFFS, dtype=jnp.int32)
    idx_all = (jnp.stack(lists, axis=2) + offs[None, None, :]).reshape(-1, 128)
    tab = _tab_call(jnp.pad(action_table, ((0, TRC - N_ACT_), (0, 0))),
                    item_table, word_table)

    x3, ue, ie = _sc_gather_call()(tab, idx_all, user_ids, item_ids,
                                   user_table, item_table)
    xl = _cast_call(x3.reshape(B, K_LIST))

    head = jnp.concatenate(
        [ue, ie, pos_item_idx.astype(jnp.float32)[:, None], price[:, None],
         list_metadata,
         jnp.zeros((B, HEAD_PAD - HEAD_USED), jnp.float32)],
        axis=1).astype(jnp.bfloat16)

    w1h = jnp.pad(W1[:HEAD_USED],
                  ((0, HEAD_PAD - HEAD_USED), (0, 0))).astype(jnp.bfloat16)
    w1p = W1[HEAD_USED:].astype(jnp.bfloat16)

    return _mlp_call(head, xl, w1h, w1p, b1.reshape(1, H_REAL), W2,
                     b2.reshape(1, F), Wo, bo.reshape(1, 1))
